# baseline (speedup 1.0000x reference)
import functools

import jax
import jax.numpy as jnp
from jax import lax
from jax.experimental import pallas as pl
from jax.experimental.pallas import tpu as pltpu
from jax.experimental.pallas import tpu_sc as plsc

N_INST = 8192
D_IN = 1024
DIM = 128
B = 128
K = 65536
T = 0.07
EMA = 0.999

ROWS = 512
COLS = 2048


def _enc_agg_body(imq_ref, imk_ref, wq_ref, wk_ref, v_ref, u_ref, wa_ref,
                  batch_ref, wcls_ref,
                  sfq_ref, sfk_ref, sqo_ref, sko_ref,
                  attq_ref, attk_ref, yprob_ref, q_ref, k_ref,
                  lpos_ref, feaq_s, feak_s, sq_s, sk_s):
    i = pl.program_id(0)
    wq = wq_ref[...]
    wk = EMA * wk_ref[...] + (1.0 - EMA) * wq
    hq = jnp.dot(imq_ref[...], wq, preferred_element_type=jnp.float32)
    hk = jnp.dot(imk_ref[...], wk, preferred_element_type=jnp.float32)
    feaq = jnp.tanh(hq[:, :DIM])
    sfq = jnp.tanh(hq[:, DIM:])
    feak = jnp.tanh(hk[:, :DIM])
    sfk = jnp.tanh(hk[:, DIM:])
    sfq_ref[...] = sfq
    sfk_ref[...] = sfk
    v = v_ref[...]
    u = u_ref[...]
    wa = wa_ref[...]
    aq = jnp.tanh(jnp.dot(feaq, v, preferred_element_type=jnp.float32)) * \
        jax.nn.sigmoid(jnp.dot(feaq, u, preferred_element_type=jnp.float32))
    ak = jnp.tanh(jnp.dot(feak, v, preferred_element_type=jnp.float32)) * \
        jax.nn.sigmoid(jnp.dot(feak, u, preferred_element_type=jnp.float32))
    sq = jnp.dot(aq, wa, preferred_element_type=jnp.float32)
    sk = jnp.dot(ak, wa, preferred_element_type=jnp.float32)
    sqo_ref[...] = sq
    sko_ref[...] = sk
    base = i * ROWS
    feaq_s[pl.ds(base, ROWS), :] = feaq
    feak_s[pl.ds(base, ROWS), :] = feak
    sq_s[pl.ds(base, ROWS), :] = sq
    sk_s[pl.ds(base, ROWS), :] = sk

    @pl.when(i == (N_INST // ROWS) - 1)
    def _():
        batch = batch_ref[...]
        seg = lax.broadcasted_iota(jnp.int32, (1, B), 1)
        onehot_b = batch == seg
        onehot = onehot_b.astype(jnp.float32)

        def branch(fea, s):
            sm = jnp.max(jnp.where(onehot_b, s, -1e30), axis=0, keepdims=True)
            sm = jnp.where(sm > -1e29, sm, 0.0)
            srow = jnp.sum(onehot * sm, axis=1, keepdims=True)
            e = jnp.exp(s - srow)
            denom = lax.dot_general(e, onehot, (((0,), (0,)), ((), ())),
                                    preferred_element_type=jnp.float32)
            drow = jnp.sum(onehot * denom, axis=1, keepdims=True)
            att = e / (drow + 1e-9)
            bagf = lax.dot_general(onehot, att * fea,
                                   (((0,), (0,)), ((), ())),
                                   preferred_element_type=jnp.float32)
            nrm = jnp.sqrt(jnp.sum(bagf * bagf, axis=1, keepdims=True))
            return att, bagf, bagf / (nrm + 1e-12)

        attq, bagfq, qn = branch(feaq_s[...], sq_s[...])
        attk, _, kn = branch(feak_s[...], sk_s[...])
        attq_ref[...] = attq
        attk_ref[...] = attk
        yprob_ref[...] = jax.nn.sigmoid(
            jnp.dot(bagfq, wcls_ref[...], preferred_element_type=jnp.float32))
        q_ref[...] = qn
        k_ref[...] = kn
        lpos_ref[...] = jnp.sum(qn * kn, axis=1, keepdims=True) / T


def _bank_body(q_ref, lpos_ref, lab_ref, bl_ref, bank_ref,
               logits_ref, carry):
    j = pl.program_id(0)
    nb = K // COLS

    @pl.when(j < nb)
    def _():
        qm = q_ref[...]
        bank_t = bank_ref[...]
        ln = jnp.dot(qm, bank_t, preferred_element_type=jnp.float32)
        bl = bl_ref[0]
        mask = lab_ref[...] == bl
        ln = jnp.where(mask, -1e9, ln) / T
        head = jnp.where(j == 0, lpos_ref[...], carry[...])
        logits_ref[...] = jnp.concatenate([head, ln[:, :COLS - 1]], axis=1)
        carry[...] = ln[:, COLS - 1:COLS]

    @pl.when(j == nb)
    def _():
        logits_ref[:, 0:1] = carry[...]


_SC_NC = 2
_SC_NS = 16
_SC_NW = _SC_NC * _SC_NS
_GRP = 128
_NGRP = K // _GRP


def _sc_bank_body(bank_hbm, qf_hbm, idx_hbm, nb_hbm,
                  patch_v, qrow_v, idx_s, sem_copy, sem_q, sem_p):
    c = lax.axis_index("c")
    s = lax.axis_index("s")
    wid = s * _SC_NC + c
    gpw = _NGRP // _SC_NW
    slab = gpw * _GRP
    h = pltpu.async_copy(bank_hbm.at[:, pl.ds(wid * slab, slab)],
                         nb_hbm.at[:, pl.ds(wid * slab, slab)], sem_copy)
    pltpu.async_copy(idx_hbm, idx_s, sem_q).wait()
    h.wait()

    def chunk_body(cidx, _):
        vec = idx_s[pl.ds(cidx * 16, 16)]
        for t in range(16):
            idx = vec[t]
            j = cidx * 16 + t
            g = idx // _GRP
            owner = g // gpw

            @pl.when(owner == wid)
            def _(idx=idx, g=g, j=j):
                cl = idx - g * _GRP
                cc = (cl // 16) * 16
                pltpu.async_copy(nb_hbm.at[:, pl.ds(g * _GRP, _GRP)],
                                 patch_v, sem_p).wait()
                pltpu.async_copy(qf_hbm.at[pl.ds(j * DIM, DIM)],
                                 qrow_v, sem_p).wait()
                m = (lax.broadcasted_iota(jnp.int32, (16,), 0) + cc) == cl

                def rc_body(rc, _):
                    qchunk = qrow_v[pl.ds(rc * 16, 16)]
                    for rr in range(16):
                        row = rc * 16 + rr
                        v = patch_v[row, pl.ds(cc, 16)]
                        vals = jnp.full((16,), qchunk[rr], jnp.float32)
                        patch_v[row, pl.ds(cc, 16)] = jnp.where(m, vals, v)
                    return 0

                lax.fori_loop(0, DIM // 16, rc_body, 0)
                pltpu.async_copy(patch_v,
                                 nb_hbm.at[:, pl.ds(g * _GRP, _GRP)],
                                 sem_p).wait()
        return 0

    lax.fori_loop(0, B // 16, chunk_body, 0)


def _sc_new_bank(bank, qn, bag_idx):
    mesh = plsc.VectorSubcoreMesh(core_axis_name="c", subcore_axis_name="s",
                                  num_cores=_SC_NC, num_subcores=_SC_NS)
    return pl.kernel(
        _sc_bank_body,
        out_type=jax.ShapeDtypeStruct((DIM, K), jnp.float32),
        mesh=mesh,
        scratch_types=[
            pltpu.VMEM((DIM, _GRP), jnp.float32),
            pltpu.VMEM((DIM,), jnp.float32),
            pltpu.VMEM((B,), jnp.int32),
            pltpu.SemaphoreType.DMA,
            pltpu.SemaphoreType.DMA,
            pltpu.SemaphoreType.DMA,
        ],
    )(bank, qn.reshape(B * DIM), bag_idx.astype(jnp.int32))


def kernel(im_q, im_k, batch, bag_idx, label, bag_label, W_enc_q, W_self_q,
           V_q, U_q, w_att_q, W_cls_q, W_enc_k, W_self_k, bank):
    f32 = jnp.float32
    wq_cat = jnp.concatenate([W_enc_q, W_self_q], axis=1)
    wk_cat = jnp.concatenate([W_enc_k, W_self_k], axis=1)

    n_row_blocks = N_INST // ROWS
    (sfq, sfk, sq, sk, attq, attk, yprob, qn, kn, lpos) = pl.pallas_call(
        _enc_agg_body,
        grid=(n_row_blocks,),
        in_specs=[
            pl.BlockSpec((ROWS, D_IN), lambda i: (i, 0)),
            pl.BlockSpec((ROWS, D_IN), lambda i: (i, 0)),
            pl.BlockSpec((D_IN, 2 * DIM), lambda i: (0, 0)),
            pl.BlockSpec((D_IN, 2 * DIM), lambda i: (0, 0)),
            pl.BlockSpec((DIM, DIM), lambda i: (0, 0)),
            pl.BlockSpec((DIM, DIM), lambda i: (0, 0)),
            pl.BlockSpec((DIM, 1), lambda i: (0, 0)),
            pl.BlockSpec((N_INST, 1), lambda i: (0, 0)),
            pl.BlockSpec((DIM, 1), lambda i: (0, 0)),
        ],
        out_specs=[
            pl.BlockSpec((ROWS, DIM), lambda i: (i, 0)),
            pl.BlockSpec((ROWS, DIM), lambda i: (i, 0)),
            pl.BlockSpec((ROWS, 1), lambda i: (i, 0)),
            pl.BlockSpec((ROWS, 1), lambda i: (i, 0)),
            pl.BlockSpec((N_INST, 1), lambda i: (0, 0)),
            pl.BlockSpec((N_INST, 1), lambda i: (0, 0)),
            pl.BlockSpec((B, 1), lambda i: (0, 0)),
            pl.BlockSpec((B, DIM), lambda i: (0, 0)),
            pl.BlockSpec((B, DIM), lambda i: (0, 0)),
            pl.BlockSpec((B, 1), lambda i: (0, 0)),
        ],
        out_shape=[
            jax.ShapeDtypeStruct((N_INST, DIM), f32),
            jax.ShapeDtypeStruct((N_INST, DIM), f32),
            jax.ShapeDtypeStruct((N_INST, 1), f32),
            jax.ShapeDtypeStruct((N_INST, 1), f32),
            jax.ShapeDtypeStruct((N_INST, 1), f32),
            jax.ShapeDtypeStruct((N_INST, 1), f32),
            jax.ShapeDtypeStruct((B, 1), f32),
            jax.ShapeDtypeStruct((B, DIM), f32),
            jax.ShapeDtypeStruct((B, DIM), f32),
            jax.ShapeDtypeStruct((B, 1), f32),
        ],
        scratch_shapes=[
            pltpu.VMEM((N_INST, DIM), f32),
            pltpu.VMEM((N_INST, DIM), f32),
            pltpu.VMEM((N_INST, 1), f32),
            pltpu.VMEM((N_INST, 1), f32),
        ],
    )(im_q, im_k, wq_cat, wk_cat, V_q, U_q, w_att_q,
      batch.reshape(N_INST, 1).astype(jnp.int32), W_cls_q)

    nbank = _sc_new_bank(bank, qn, bag_idx)

    n_col_blocks = K // COLS
    last = n_col_blocks - 1
    logits = pl.pallas_call(
        _bank_body,
        grid=(n_col_blocks + 1,),
        in_specs=[
            pl.BlockSpec((B, DIM), lambda j: (0, 0)),
            pl.BlockSpec((B, 1), lambda j: (0, 0)),
            pl.BlockSpec((B, 1), lambda j: (0, 0)),
            pl.BlockSpec((1, 1, COLS), lambda j: (jnp.minimum(j, last), 0, 0)),
            pl.BlockSpec((DIM, COLS), lambda j: (0, jnp.minimum(j, last))),
        ],
        out_specs=pl.BlockSpec((B, COLS), lambda j: (0, j)),
        out_shape=jax.ShapeDtypeStruct((B, K + 1), f32),
        scratch_shapes=[
            pltpu.VMEM((B, 1), f32),
        ],
    )(qn, lpos, label.reshape(B, 1).astype(jnp.int32),
      bag_label.reshape(n_col_blocks, 1, COLS).astype(jnp.int32),
      bank)

    labels = jnp.zeros((B,), jnp.int32)
    return (yprob, logits, labels, nbank, sfq, sfk,
            attq.reshape(N_INST), attk.reshape(N_INST),
            sq.reshape(N_INST), sk.reshape(N_INST))


# speedup vs baseline: 1.0099x; 1.0099x over previous
import functools

import jax
import jax.numpy as jnp
from jax import lax
from jax.experimental import pallas as pl
from jax.experimental.pallas import tpu as pltpu
from jax.experimental.pallas import tpu_sc as plsc

N_INST = 8192
D_IN = 1024
DIM = 128
B = 128
K = 65536
T = 0.07
EMA = 0.999

ROWS = 512
COLS = 2048


def _enc_agg_body(imq_ref, imk_ref, wq_ref, wk_ref, v_ref, u_ref, wa_ref,
                  batch_ref, wcls_ref,
                  sfq_ref, sfk_ref, sqo_ref, sko_ref,
                  attq_ref, attk_ref, yprob_ref, q_ref, k_ref,
                  lpos_ref, feaq_s, feak_s, sq_s, sk_s):
    i = pl.program_id(0)
    wq = wq_ref[...]
    wk = EMA * wk_ref[...] + (1.0 - EMA) * wq
    hq = jnp.dot(imq_ref[...], wq, preferred_element_type=jnp.float32)
    hk = jnp.dot(imk_ref[...], wk, preferred_element_type=jnp.float32)
    feaq = jnp.tanh(hq[:, :DIM])
    sfq = jnp.tanh(hq[:, DIM:])
    feak = jnp.tanh(hk[:, :DIM])
    sfk = jnp.tanh(hk[:, DIM:])
    sfq_ref[...] = sfq
    sfk_ref[...] = sfk
    v = v_ref[...]
    u = u_ref[...]
    wa = wa_ref[...]
    aq = jnp.tanh(jnp.dot(feaq, v, preferred_element_type=jnp.float32)) * \
        jax.nn.sigmoid(jnp.dot(feaq, u, preferred_element_type=jnp.float32))
    ak = jnp.tanh(jnp.dot(feak, v, preferred_element_type=jnp.float32)) * \
        jax.nn.sigmoid(jnp.dot(feak, u, preferred_element_type=jnp.float32))
    sq = jnp.dot(aq, wa, preferred_element_type=jnp.float32)
    sk = jnp.dot(ak, wa, preferred_element_type=jnp.float32)
    sqo_ref[...] = sq
    sko_ref[...] = sk
    base = i * ROWS
    feaq_s[pl.ds(base, ROWS), :] = feaq
    feak_s[pl.ds(base, ROWS), :] = feak
    sq_s[pl.ds(base, ROWS), :] = sq
    sk_s[pl.ds(base, ROWS), :] = sk

    @pl.when(i == (N_INST // ROWS) - 1)
    def _():
        batch = batch_ref[...]
        seg = lax.broadcasted_iota(jnp.int32, (1, B), 1)
        onehot_b = batch == seg
        onehot = onehot_b.astype(jnp.float32)

        def branch(fea, s):
            sm = jnp.max(jnp.where(onehot_b, s, -1e30), axis=0, keepdims=True)
            sm = jnp.where(sm > -1e29, sm, 0.0)
            srow = jnp.sum(onehot * sm, axis=1, keepdims=True)
            e = jnp.exp(s - srow)
            denom = lax.dot_general(e, onehot, (((0,), (0,)), ((), ())),
                                    preferred_element_type=jnp.float32)
            drow = jnp.sum(onehot * denom, axis=1, keepdims=True)
            att = e / (drow + 1e-9)
            bagf = lax.dot_general(onehot, att * fea,
                                   (((0,), (0,)), ((), ())),
                                   preferred_element_type=jnp.float32)
            nrm = jnp.sqrt(jnp.sum(bagf * bagf, axis=1, keepdims=True))
            return att, bagf, bagf / (nrm + 1e-12)

        attq, bagfq, qn = branch(feaq_s[...], sq_s[...])
        attk, _, kn = branch(feak_s[...], sk_s[...])
        attq_ref[...] = attq
        attk_ref[...] = attk
        yprob_ref[...] = jax.nn.sigmoid(
            jnp.dot(bagfq, wcls_ref[...], preferred_element_type=jnp.float32))
        q_ref[...] = qn
        k_ref[...] = kn
        lpos_ref[...] = jnp.sum(qn * kn, axis=1, keepdims=True) / T


def _bank_body(q_ref, lpos_ref, lab_ref, bl_ref, bank_ref,
               logits_ref, carry):
    j = pl.program_id(0)
    nb = K // COLS

    @pl.when(j < nb)
    def _():
        qm = q_ref[...]
        bank_t = bank_ref[...]
        ln = jnp.dot(qm, bank_t, preferred_element_type=jnp.float32)
        bl = bl_ref[0]
        mask = lab_ref[...] == bl
        ln = jnp.where(mask, -1e9, ln) / T
        head = jnp.where(j == 0, lpos_ref[...], carry[...])
        logits_ref[...] = jnp.concatenate([head, ln[:, :COLS - 1]], axis=1)
        carry[...] = ln[:, COLS - 1:COLS]

    @pl.when(j == nb)
    def _():
        logits_ref[:, 0:1] = carry[...]


_SC_NC = 2
_SC_NS = 16
_SC_NW = _SC_NC * _SC_NS
_GRP = 128
_NGRP = K // _GRP


def _sc_bank_body(bank_hbm, qf_hbm, idx_hbm, nb_hbm,
                  patch_v, qrow_v, idx_s, sem_copy, sem_q, sem_p):
    c = lax.axis_index("c")
    s = lax.axis_index("s")
    wid = s * _SC_NC + c
    gpw = _NGRP // _SC_NW
    slab = gpw * _GRP
    nchunk = 8
    cw = slab // nchunk
    hs = []
    for u in range(nchunk):
        off = wid * slab + u * cw
        hs.append(pltpu.async_copy(bank_hbm.at[:, pl.ds(off, cw)],
                                   nb_hbm.at[:, pl.ds(off, cw)], sem_copy))
    pltpu.async_copy(idx_hbm, idx_s, sem_q).wait()
    for h in hs:
        h.wait()

    def chunk_body(cidx, _):
        vec = idx_s[pl.ds(cidx * 16, 16)]
        for t in range(16):
            idx = vec[t]
            j = cidx * 16 + t
            g = idx // _GRP
            owner = g // gpw

            @pl.when(owner == wid)
            def _(idx=idx, g=g, j=j):
                cl = idx - g * _GRP
                cc = (cl // 16) * 16
                pltpu.async_copy(nb_hbm.at[:, pl.ds(g * _GRP, _GRP)],
                                 patch_v, sem_p).wait()
                pltpu.async_copy(qf_hbm.at[pl.ds(j * DIM, DIM)],
                                 qrow_v, sem_p).wait()
                m = (lax.broadcasted_iota(jnp.int32, (16,), 0) + cc) == cl

                def rc_body(rc, _):
                    qchunk = qrow_v[pl.ds(rc * 16, 16)]
                    for rr in range(16):
                        row = rc * 16 + rr
                        v = patch_v[row, pl.ds(cc, 16)]
                        vals = jnp.full((16,), qchunk[rr], jnp.float32)
                        patch_v[row, pl.ds(cc, 16)] = jnp.where(m, vals, v)
                    return 0

                lax.fori_loop(0, DIM // 16, rc_body, 0)
                pltpu.async_copy(patch_v,
                                 nb_hbm.at[:, pl.ds(g * _GRP, _GRP)],
                                 sem_p).wait()
        return 0

    if False:
        lax.fori_loop(0, B // 16, chunk_body, 0)


def _sc_new_bank(bank, qn, bag_idx):
    mesh = plsc.VectorSubcoreMesh(core_axis_name="c", subcore_axis_name="s",
                                  num_cores=_SC_NC, num_subcores=_SC_NS)
    return pl.kernel(
        _sc_bank_body,
        out_type=jax.ShapeDtypeStruct((DIM, K), jnp.float32),
        mesh=mesh,
        scratch_types=[
            pltpu.VMEM((DIM, _GRP), jnp.float32),
            pltpu.VMEM((DIM,), jnp.float32),
            pltpu.VMEM((B,), jnp.int32),
            pltpu.SemaphoreType.DMA,
            pltpu.SemaphoreType.DMA,
            pltpu.SemaphoreType.DMA,
        ],
    )(bank, qn.reshape(B * DIM), bag_idx.astype(jnp.int32))


def kernel(im_q, im_k, batch, bag_idx, label, bag_label, W_enc_q, W_self_q,
           V_q, U_q, w_att_q, W_cls_q, W_enc_k, W_self_k, bank):
    f32 = jnp.float32
    wq_cat = jnp.concatenate([W_enc_q, W_self_q], axis=1)
    wk_cat = jnp.concatenate([W_enc_k, W_self_k], axis=1)

    n_row_blocks = N_INST // ROWS
    (sfq, sfk, sq, sk, attq, attk, yprob, qn, kn, lpos) = pl.pallas_call(
        _enc_agg_body,
        grid=(n_row_blocks,),
        in_specs=[
            pl.BlockSpec((ROWS, D_IN), lambda i: (i, 0)),
            pl.BlockSpec((ROWS, D_IN), lambda i: (i, 0)),
            pl.BlockSpec((D_IN, 2 * DIM), lambda i: (0, 0)),
            pl.BlockSpec((D_IN, 2 * DIM), lambda i: (0, 0)),
            pl.BlockSpec((DIM, DIM), lambda i: (0, 0)),
            pl.BlockSpec((DIM, DIM), lambda i: (0, 0)),
            pl.BlockSpec((DIM, 1), lambda i: (0, 0)),
            pl.BlockSpec((N_INST, 1), lambda i: (0, 0)),
            pl.BlockSpec((DIM, 1), lambda i: (0, 0)),
        ],
        out_specs=[
            pl.BlockSpec((ROWS, DIM), lambda i: (i, 0)),
            pl.BlockSpec((ROWS, DIM), lambda i: (i, 0)),
            pl.BlockSpec((ROWS, 1), lambda i: (i, 0)),
            pl.BlockSpec((ROWS, 1), lambda i: (i, 0)),
            pl.BlockSpec((N_INST, 1), lambda i: (0, 0)),
            pl.BlockSpec((N_INST, 1), lambda i: (0, 0)),
            pl.BlockSpec((B, 1), lambda i: (0, 0)),
            pl.BlockSpec((B, DIM), lambda i: (0, 0)),
            pl.BlockSpec((B, DIM), lambda i: (0, 0)),
            pl.BlockSpec((B, 1), lambda i: (0, 0)),
        ],
        out_shape=[
            jax.ShapeDtypeStruct((N_INST, DIM), f32),
            jax.ShapeDtypeStruct((N_INST, DIM), f32),
            jax.ShapeDtypeStruct((N_INST, 1), f32),
            jax.ShapeDtypeStruct((N_INST, 1), f32),
            jax.ShapeDtypeStruct((N_INST, 1), f32),
            jax.ShapeDtypeStruct((N_INST, 1), f32),
            jax.ShapeDtypeStruct((B, 1), f32),
            jax.ShapeDtypeStruct((B, DIM), f32),
            jax.ShapeDtypeStruct((B, DIM), f32),
            jax.ShapeDtypeStruct((B, 1), f32),
        ],
        scratch_shapes=[
            pltpu.VMEM((N_INST, DIM), f32),
            pltpu.VMEM((N_INST, DIM), f32),
            pltpu.VMEM((N_INST, 1), f32),
            pltpu.VMEM((N_INST, 1), f32),
        ],
    )(im_q, im_k, wq_cat, wk_cat, V_q, U_q, w_att_q,
      batch.reshape(N_INST, 1).astype(jnp.int32), W_cls_q)

    nbank = _sc_new_bank(bank, qn, bag_idx)

    n_col_blocks = K // COLS
    last = n_col_blocks - 1
    logits = pl.pallas_call(
        _bank_body,
        grid=(n_col_blocks + 1,),
        in_specs=[
            pl.BlockSpec((B, DIM), lambda j: (0, 0)),
            pl.BlockSpec((B, 1), lambda j: (0, 0)),
            pl.BlockSpec((B, 1), lambda j: (0, 0)),
            pl.BlockSpec((1, 1, COLS), lambda j: (jnp.minimum(j, last), 0, 0)),
            pl.BlockSpec((DIM, COLS), lambda j: (0, jnp.minimum(j, last))),
        ],
        out_specs=pl.BlockSpec((B, COLS), lambda j: (0, j)),
        out_shape=jax.ShapeDtypeStruct((B, K + 1), f32),
        scratch_shapes=[
            pltpu.VMEM((B, 1), f32),
        ],
    )(qn, lpos, label.reshape(B, 1).astype(jnp.int32),
      bag_label.reshape(n_col_blocks, 1, COLS).astype(jnp.int32),
      bank)

    labels = jnp.zeros((B,), jnp.int32)
    return (yprob, logits, labels, nbank, sfq, sfk,
            attq.reshape(N_INST), attk.reshape(N_INST),
            sq.reshape(N_INST), sk.reshape(N_INST))


# speedup vs baseline: 6.9146x; 6.8468x over previous
import jax
import jax.numpy as jnp
from jax import lax
from jax.experimental import pallas as pl
from jax.experimental.pallas import tpu as pltpu

N_INST = 8192
D_IN = 1024
DIM = 128
B = 128
K = 65536
T = 0.07
EMA = 0.999

ROWS = 512
COLS = 2048


def _enc_agg_body(imq_ref, imk_ref, wq_ref, wk_ref, v_ref, u_ref, wa_ref,
                  batch_ref, wcls_ref,
                  sfq_ref, sfk_ref, sqo_ref, sko_ref,
                  attq_ref, attk_ref, yprob_ref, q_ref, k_ref, lpos_ref,
                  feaq_s, feak_s, sq_s, sk_s):
    i = pl.program_id(0)
    wq = wq_ref[...]
    wk = EMA * wk_ref[...] + (1.0 - EMA) * wq
    hq = jnp.dot(imq_ref[...], wq, preferred_element_type=jnp.float32)
    hk = jnp.dot(imk_ref[...], wk, preferred_element_type=jnp.float32)
    feaq = jnp.tanh(hq[:, :DIM])
    sfq = jnp.tanh(hq[:, DIM:])
    feak = jnp.tanh(hk[:, :DIM])
    sfk = jnp.tanh(hk[:, DIM:])
    sfq_ref[...] = sfq
    sfk_ref[...] = sfk
    v = v_ref[...]
    u = u_ref[...]
    wa = wa_ref[...]
    aq = jnp.tanh(jnp.dot(feaq, v, preferred_element_type=jnp.float32)) * \
        jax.nn.sigmoid(jnp.dot(feaq, u, preferred_element_type=jnp.float32))
    ak = jnp.tanh(jnp.dot(feak, v, preferred_element_type=jnp.float32)) * \
        jax.nn.sigmoid(jnp.dot(feak, u, preferred_element_type=jnp.float32))
    sq = jnp.dot(aq, wa, preferred_element_type=jnp.float32)
    sk = jnp.dot(ak, wa, preferred_element_type=jnp.float32)
    sqo_ref[...] = sq
    sko_ref[...] = sk
    base = i * ROWS
    feaq_s[pl.ds(base, ROWS), :] = feaq
    feak_s[pl.ds(base, ROWS), :] = feak
    sq_s[pl.ds(base, ROWS), :] = sq
    sk_s[pl.ds(base, ROWS), :] = sk

    @pl.when(i == (N_INST // ROWS) - 1)
    def _():
        batch = batch_ref[...]
        seg = lax.broadcasted_iota(jnp.int32, (1, B), 1)
        onehot_b = batch == seg
        onehot = onehot_b.astype(jnp.float32)

        def branch(fea, s):
            sm = jnp.max(jnp.where(onehot_b, s, -1e30), axis=0, keepdims=True)
            sm = jnp.where(sm > -1e29, sm, 0.0)
            srow = jnp.sum(onehot * sm, axis=1, keepdims=True)
            e = jnp.exp(s - srow)
            denom = lax.dot_general(e, onehot, (((0,), (0,)), ((), ())),
                                    preferred_element_type=jnp.float32)
            drow = jnp.sum(onehot * denom, axis=1, keepdims=True)
            att = e / (drow + 1e-9)
            bagf = lax.dot_general(onehot, att * fea,
                                   (((0,), (0,)), ((), ())),
                                   preferred_element_type=jnp.float32)
            nrm = jnp.sqrt(jnp.sum(bagf * bagf, axis=1, keepdims=True))
            return att, bagf, bagf / (nrm + 1e-12)

        attq, bagfq, qn = branch(feaq_s[...], sq_s[...])
        attk, _, kn = branch(feak_s[...], sk_s[...])
        attq_ref[...] = attq
        attk_ref[...] = attk
        yprob_ref[...] = jax.nn.sigmoid(
            jnp.dot(bagfq, wcls_ref[...], preferred_element_type=jnp.float32))
        q_ref[...] = qn
        k_ref[...] = kn
        lpos_ref[...] = jnp.sum(qn * kn, axis=1, keepdims=True) / T


def _bank_body(q_ref, lpos_ref, lab_ref, bl_ref, bic_ref, bir_ref, bank_ref,
               logits_ref, nbank_ref, carry):
    j = pl.program_id(0)
    nb = K // COLS

    @pl.when(j < nb)
    def _():
        qm = q_ref[...]
        bank_t = bank_ref[...]
        ln = jnp.dot(qm, bank_t, preferred_element_type=jnp.float32)
        bl = bl_ref[0]
        mask = lab_ref[...] == bl
        ln = jnp.where(mask, -1e9, ln) / T
        head = jnp.where(j == 0, lpos_ref[...], carry[...])
        logits_ref[...] = jnp.concatenate([head, ln[:, :COLS - 1]], axis=1)
        carry[...] = ln[:, COLS - 1:COLS]
        bic = bic_ref[...]
        bir = bir_ref[...]
        ir = lax.broadcasted_iota(jnp.int32, (1, B), 1)
        ic = lax.broadcasted_iota(jnp.int32, (B, 1), 0)
        dup_later = (bic == bir) & (ir > ic)
        is_last = jnp.max(dup_later.astype(jnp.int32), axis=1,
                          keepdims=True) == 0
        cols = lax.broadcasted_iota(jnp.int32, (B, COLS), 1) + j * COLS
        sel = ((bic == cols) & is_last).astype(jnp.float32)
        hit = jnp.max(sel, axis=0, keepdims=True)
        over = lax.dot_general(qm, sel, (((0,), (0,)), ((), ())),
                               preferred_element_type=jnp.float32)
        nbank_ref[...] = bank_t * (1.0 - hit) + over

    @pl.when(j == nb)
    def _():
        logits_ref[:, 0:1] = carry[...]


def kernel(im_q, im_k, batch, bag_idx, label, bag_label, W_enc_q, W_self_q,
           V_q, U_q, w_att_q, W_cls_q, W_enc_k, W_self_k, bank):
    f32 = jnp.float32
    wq_cat = jnp.concatenate([W_enc_q, W_self_q], axis=1)
    wk_cat = jnp.concatenate([W_enc_k, W_self_k], axis=1)

    n_row_blocks = N_INST // ROWS
    (sfq, sfk, sq, sk, attq, attk, yprob, qn, kn, lpos) = pl.pallas_call(
        _enc_agg_body,
        grid=(n_row_blocks,),
        in_specs=[
            pl.BlockSpec((ROWS, D_IN), lambda i: (i, 0)),
            pl.BlockSpec((ROWS, D_IN), lambda i: (i, 0)),
            pl.BlockSpec((D_IN, 2 * DIM), lambda i: (0, 0)),
            pl.BlockSpec((D_IN, 2 * DIM), lambda i: (0, 0)),
            pl.BlockSpec((DIM, DIM), lambda i: (0, 0)),
            pl.BlockSpec((DIM, DIM), lambda i: (0, 0)),
            pl.BlockSpec((DIM, 1), lambda i: (0, 0)),
            pl.BlockSpec((N_INST, 1), lambda i: (0, 0)),
            pl.BlockSpec((DIM, 1), lambda i: (0, 0)),
        ],
        out_specs=[
            pl.BlockSpec((ROWS, DIM), lambda i: (i, 0)),
            pl.BlockSpec((ROWS, DIM), lambda i: (i, 0)),
            pl.BlockSpec((ROWS, 1), lambda i: (i, 0)),
            pl.BlockSpec((ROWS, 1), lambda i: (i, 0)),
            pl.BlockSpec((N_INST, 1), lambda i: (0, 0)),
            pl.BlockSpec((N_INST, 1), lambda i: (0, 0)),
            pl.BlockSpec((B, 1), lambda i: (0, 0)),
            pl.BlockSpec((B, DIM), lambda i: (0, 0)),
            pl.BlockSpec((B, DIM), lambda i: (0, 0)),
            pl.BlockSpec((B, 1), lambda i: (0, 0)),
        ],
        out_shape=[
            jax.ShapeDtypeStruct((N_INST, DIM), f32),
            jax.ShapeDtypeStruct((N_INST, DIM), f32),
            jax.ShapeDtypeStruct((N_INST, 1), f32),
            jax.ShapeDtypeStruct((N_INST, 1), f32),
            jax.ShapeDtypeStruct((N_INST, 1), f32),
            jax.ShapeDtypeStruct((N_INST, 1), f32),
            jax.ShapeDtypeStruct((B, 1), f32),
            jax.ShapeDtypeStruct((B, DIM), f32),
            jax.ShapeDtypeStruct((B, DIM), f32),
            jax.ShapeDtypeStruct((B, 1), f32),
        ],
        scratch_shapes=[
            pltpu.VMEM((N_INST, DIM), f32),
            pltpu.VMEM((N_INST, DIM), f32),
            pltpu.VMEM((N_INST, 1), f32),
            pltpu.VMEM((N_INST, 1), f32),
        ],
    )(im_q, im_k, wq_cat, wk_cat, V_q, U_q, w_att_q,
      batch.reshape(N_INST, 1).astype(jnp.int32), W_cls_q)

    n_col_blocks = K // COLS
    last = n_col_blocks - 1
    logits, nbank = pl.pallas_call(
        _bank_body,
        grid=(n_col_blocks + 1,),
        in_specs=[
            pl.BlockSpec((B, DIM), lambda j: (0, 0)),
            pl.BlockSpec((B, 1), lambda j: (0, 0)),
            pl.BlockSpec((B, 1), lambda j: (0, 0)),
            pl.BlockSpec((1, 1, COLS), lambda j: (jnp.minimum(j, last), 0, 0)),
            pl.BlockSpec((B, 1), lambda j: (0, 0)),
            pl.BlockSpec((1, B), lambda j: (0, 0)),
            pl.BlockSpec((DIM, COLS), lambda j: (0, jnp.minimum(j, last))),
        ],
        out_specs=[
            pl.BlockSpec((B, COLS), lambda j: (0, j)),
            pl.BlockSpec((DIM, COLS), lambda j: (0, jnp.minimum(j, last))),
        ],
        out_shape=[
            jax.ShapeDtypeStruct((B, K + 1), f32),
            jax.ShapeDtypeStruct((DIM, K), f32),
        ],
        scratch_shapes=[
            pltpu.VMEM((B, 1), f32),
        ],
    )(qn, lpos, label.reshape(B, 1).astype(jnp.int32),
      bag_label.reshape(n_col_blocks, 1, COLS).astype(jnp.int32),
      bag_idx.reshape(B, 1).astype(jnp.int32),
      bag_idx.reshape(1, B).astype(jnp.int32), bank)

    labels = jnp.zeros((B,), jnp.int32)
    return (yprob, logits, labels, nbank, sfq, sfk,
            attq.reshape(N_INST), attk.reshape(N_INST),
            sq.reshape(N_INST), sk.reshape(N_INST))


# speedup vs baseline: 7.4123x; 1.0720x over previous
import jax
import jax.numpy as jnp
from jax import lax
from jax.experimental import pallas as pl
from jax.experimental.pallas import tpu as pltpu

N_INST = 8192
D_IN = 1024
DIM = 128
B = 128
K = 65536
T = 0.07
EMA = 0.999

ROWS = 512
COLS = 4096


def _enc_agg_body(imq_ref, imk_ref, wq_ref, wk_ref, v_ref, u_ref, wa_ref,
                  batch_ref, wcls_ref,
                  sfq_ref, sfk_ref, sqo_ref, sko_ref,
                  attq_ref, attk_ref, yprob_ref, q_ref, k_ref, lpos_ref,
                  feaq_s, feak_s, sq_s, sk_s):
    i = pl.program_id(0)
    wq = wq_ref[...]
    wk = EMA * wk_ref[...] + (1.0 - EMA) * wq
    hq = jnp.dot(imq_ref[...], wq, preferred_element_type=jnp.float32)
    hk = jnp.dot(imk_ref[...], wk, preferred_element_type=jnp.float32)
    feaq = jnp.tanh(hq[:, :DIM])
    sfq = jnp.tanh(hq[:, DIM:])
    feak = jnp.tanh(hk[:, :DIM])
    sfk = jnp.tanh(hk[:, DIM:])
    sfq_ref[...] = sfq
    sfk_ref[...] = sfk
    v = v_ref[...]
    u = u_ref[...]
    wa = wa_ref[...]
    aq = jnp.tanh(jnp.dot(feaq, v, preferred_element_type=jnp.float32)) * \
        jax.nn.sigmoid(jnp.dot(feaq, u, preferred_element_type=jnp.float32))
    ak = jnp.tanh(jnp.dot(feak, v, preferred_element_type=jnp.float32)) * \
        jax.nn.sigmoid(jnp.dot(feak, u, preferred_element_type=jnp.float32))
    sq = jnp.dot(aq, wa, preferred_element_type=jnp.float32)
    sk = jnp.dot(ak, wa, preferred_element_type=jnp.float32)
    sqo_ref[...] = sq
    sko_ref[...] = sk
    base = i * ROWS
    feaq_s[pl.ds(base, ROWS), :] = feaq
    feak_s[pl.ds(base, ROWS), :] = feak
    sq_s[pl.ds(base, ROWS), :] = sq
    sk_s[pl.ds(base, ROWS), :] = sk

    @pl.when(i == (N_INST // ROWS) - 1)
    def _():
        batch = batch_ref[...]
        seg = lax.broadcasted_iota(jnp.int32, (1, B), 1)
        onehot_b = batch == seg
        onehot = onehot_b.astype(jnp.float32)

        def branch(fea, s):
            sm = jnp.max(jnp.where(onehot_b, s, -1e30), axis=0, keepdims=True)
            sm = jnp.where(sm > -1e29, sm, 0.0)
            srow = jnp.sum(onehot * sm, axis=1, keepdims=True)
            e = jnp.exp(s - srow)
            denom = lax.dot_general(e, onehot, (((0,), (0,)), ((), ())),
                                    preferred_element_type=jnp.float32)
            drow = jnp.sum(onehot * denom, axis=1, keepdims=True)
            att = e / (drow + 1e-9)
            bagf = lax.dot_general(onehot, att * fea,
                                   (((0,), (0,)), ((), ())),
                                   preferred_element_type=jnp.float32)
            nrm = jnp.sqrt(jnp.sum(bagf * bagf, axis=1, keepdims=True))
            return att, bagf, bagf / (nrm + 1e-12)

        attq, bagfq, qn = branch(feaq_s[...], sq_s[...])
        attk, _, kn = branch(feak_s[...], sk_s[...])
        attq_ref[...] = attq
        attk_ref[...] = attk
        yprob_ref[...] = jax.nn.sigmoid(
            jnp.dot(bagfq, wcls_ref[...], preferred_element_type=jnp.float32))
        q_ref[...] = qn
        k_ref[...] = kn
        lpos_ref[...] = jnp.sum(qn * kn, axis=1, keepdims=True) / T


def _bank_body(q_ref, lpos_ref, lab_ref, bl_ref, bic_ref, bir_ref, bank_ref,
               logits_ref, nbank_ref, carry):
    j = pl.program_id(0)
    nb = K // COLS

    @pl.when(j < nb)
    def _():
        qm = q_ref[...]
        bank_t = bank_ref[...]
        ln = jnp.dot(qm, bank_t, preferred_element_type=jnp.float32)
        bl = bl_ref[0]
        mask = lab_ref[...] == bl
        ln = jnp.where(mask, -1e9, ln) / T
        head = jnp.where(j == 0, lpos_ref[...], carry[...])
        logits_ref[...] = jnp.concatenate([head, ln[:, :COLS - 1]], axis=1)
        carry[...] = ln[:, COLS - 1:COLS]
        bic = bic_ref[...]
        bir = bir_ref[...]
        ir = lax.broadcasted_iota(jnp.int32, (1, B), 1)
        ic = lax.broadcasted_iota(jnp.int32, (B, 1), 0)
        dup_later = (bic == bir) & (ir > ic)
        is_last = jnp.max(dup_later.astype(jnp.int32), axis=1,
                          keepdims=True) == 0
        cols = lax.broadcasted_iota(jnp.int32, (B, COLS), 1) + j * COLS
        sel = ((bic == cols) & is_last).astype(jnp.float32)
        hit = jnp.max(sel, axis=0, keepdims=True)
        over = lax.dot_general(qm, sel, (((0,), (0,)), ((), ())),
                               preferred_element_type=jnp.float32)
        nbank_ref[...] = bank_t * (1.0 - hit) + over

    @pl.when(j == nb)
    def _():
        logits_ref[:, 0:1] = carry[...]


def kernel(im_q, im_k, batch, bag_idx, label, bag_label, W_enc_q, W_self_q,
           V_q, U_q, w_att_q, W_cls_q, W_enc_k, W_self_k, bank):
    f32 = jnp.float32
    wq_cat = jnp.concatenate([W_enc_q, W_self_q], axis=1)
    wk_cat = jnp.concatenate([W_enc_k, W_self_k], axis=1)

    n_row_blocks = N_INST // ROWS
    (sfq, sfk, sq, sk, attq, attk, yprob, qn, kn, lpos) = pl.pallas_call(
        _enc_agg_body,
        grid=(n_row_blocks,),
        in_specs=[
            pl.BlockSpec((ROWS, D_IN), lambda i: (i, 0)),
            pl.BlockSpec((ROWS, D_IN), lambda i: (i, 0)),
            pl.BlockSpec((D_IN, 2 * DIM), lambda i: (0, 0)),
            pl.BlockSpec((D_IN, 2 * DIM), lambda i: (0, 0)),
            pl.BlockSpec((DIM, DIM), lambda i: (0, 0)),
            pl.BlockSpec((DIM, DIM), lambda i: (0, 0)),
            pl.BlockSpec((DIM, 1), lambda i: (0, 0)),
            pl.BlockSpec((N_INST, 1), lambda i: (0, 0)),
            pl.BlockSpec((DIM, 1), lambda i: (0, 0)),
        ],
        out_specs=[
            pl.BlockSpec((ROWS, DIM), lambda i: (i, 0)),
            pl.BlockSpec((ROWS, DIM), lambda i: (i, 0)),
            pl.BlockSpec((ROWS, 1), lambda i: (i, 0)),
            pl.BlockSpec((ROWS, 1), lambda i: (i, 0)),
            pl.BlockSpec((N_INST, 1), lambda i: (0, 0)),
            pl.BlockSpec((N_INST, 1), lambda i: (0, 0)),
            pl.BlockSpec((B, 1), lambda i: (0, 0)),
            pl.BlockSpec((B, DIM), lambda i: (0, 0)),
            pl.BlockSpec((B, DIM), lambda i: (0, 0)),
            pl.BlockSpec((B, 1), lambda i: (0, 0)),
        ],
        out_shape=[
            jax.ShapeDtypeStruct((N_INST, DIM), f32),
            jax.ShapeDtypeStruct((N_INST, DIM), f32),
            jax.ShapeDtypeStruct((N_INST, 1), f32),
            jax.ShapeDtypeStruct((N_INST, 1), f32),
            jax.ShapeDtypeStruct((N_INST, 1), f32),
            jax.ShapeDtypeStruct((N_INST, 1), f32),
            jax.ShapeDtypeStruct((B, 1), f32),
            jax.ShapeDtypeStruct((B, DIM), f32),
            jax.ShapeDtypeStruct((B, DIM), f32),
            jax.ShapeDtypeStruct((B, 1), f32),
        ],
        scratch_shapes=[
            pltpu.VMEM((N_INST, DIM), f32),
            pltpu.VMEM((N_INST, DIM), f32),
            pltpu.VMEM((N_INST, 1), f32),
            pltpu.VMEM((N_INST, 1), f32),
        ],
    )(im_q, im_k, wq_cat, wk_cat, V_q, U_q, w_att_q,
      batch.reshape(N_INST, 1).astype(jnp.int32), W_cls_q)

    n_col_blocks = K // COLS
    last = n_col_blocks - 1
    logits, nbank = pl.pallas_call(
        _bank_body,
        grid=(n_col_blocks + 1,),
        in_specs=[
            pl.BlockSpec((B, DIM), lambda j: (0, 0)),
            pl.BlockSpec((B, 1), lambda j: (0, 0)),
            pl.BlockSpec((B, 1), lambda j: (0, 0)),
            pl.BlockSpec((1, 1, COLS), lambda j: (jnp.minimum(j, last), 0, 0)),
            pl.BlockSpec((B, 1), lambda j: (0, 0)),
            pl.BlockSpec((1, B), lambda j: (0, 0)),
            pl.BlockSpec((DIM, COLS), lambda j: (0, jnp.minimum(j, last))),
        ],
        out_specs=[
            pl.BlockSpec((B, COLS), lambda j: (0, j)),
            pl.BlockSpec((DIM, COLS), lambda j: (0, jnp.minimum(j, last))),
        ],
        out_shape=[
            jax.ShapeDtypeStruct((B, K + 1), f32),
            jax.ShapeDtypeStruct((DIM, K), f32),
        ],
        scratch_shapes=[
            pltpu.VMEM((B, 1), f32),
        ],
    )(qn, lpos, label.reshape(B, 1).astype(jnp.int32),
      bag_label.reshape(n_col_blocks, 1, COLS).astype(jnp.int32),
      bag_idx.reshape(B, 1).astype(jnp.int32),
      bag_idx.reshape(1, B).astype(jnp.int32), bank)

    labels = jnp.zeros((B,), jnp.int32)
    return (yprob, logits, labels, nbank, sfq, sfk,
            attq.reshape(N_INST), attk.reshape(N_INST),
            sq.reshape(N_INST), sk.reshape(N_INST))


# speedup vs baseline: 7.5852x; 1.0233x over previous
import jax
import jax.numpy as jnp
from jax import lax
from jax.experimental import pallas as pl
from jax.experimental.pallas import tpu as pltpu

N_INST = 8192
D_IN = 1024
DIM = 128
B = 128
K = 65536
T = 0.07
EMA = 0.999

ROWS = 512
COLS = 8192


def _enc_agg_body(imq_ref, imk_ref, wq_ref, wk_ref, v_ref, u_ref, wa_ref,
                  batch_ref, wcls_ref,
                  sfq_ref, sfk_ref, sqo_ref, sko_ref,
                  attq_ref, attk_ref, yprob_ref, q_ref, k_ref, lpos_ref,
                  feaq_s, feak_s, sq_s, sk_s):
    i = pl.program_id(0)
    wq = wq_ref[...]
    wk = EMA * wk_ref[...] + (1.0 - EMA) * wq
    hq = jnp.dot(imq_ref[...], wq, preferred_element_type=jnp.float32)
    hk = jnp.dot(imk_ref[...], wk, preferred_element_type=jnp.float32)
    feaq = jnp.tanh(hq[:, :DIM])
    sfq = jnp.tanh(hq[:, DIM:])
    feak = jnp.tanh(hk[:, :DIM])
    sfk = jnp.tanh(hk[:, DIM:])
    sfq_ref[...] = sfq
    sfk_ref[...] = sfk
    v = v_ref[...]
    u = u_ref[...]
    wa = wa_ref[...]
    aq = jnp.tanh(jnp.dot(feaq, v, preferred_element_type=jnp.float32)) * \
        jax.nn.sigmoid(jnp.dot(feaq, u, preferred_element_type=jnp.float32))
    ak = jnp.tanh(jnp.dot(feak, v, preferred_element_type=jnp.float32)) * \
        jax.nn.sigmoid(jnp.dot(feak, u, preferred_element_type=jnp.float32))
    sq = jnp.dot(aq, wa, preferred_element_type=jnp.float32)
    sk = jnp.dot(ak, wa, preferred_element_type=jnp.float32)
    sqo_ref[...] = sq
    sko_ref[...] = sk
    base = i * ROWS
    feaq_s[pl.ds(base, ROWS), :] = feaq
    feak_s[pl.ds(base, ROWS), :] = feak
    sq_s[pl.ds(base, ROWS), :] = sq
    sk_s[pl.ds(base, ROWS), :] = sk

    @pl.when(i == (N_INST // ROWS) - 1)
    def _():
        batch = batch_ref[...]
        seg = lax.broadcasted_iota(jnp.int32, (1, B), 1)
        onehot_b = batch == seg
        onehot = onehot_b.astype(jnp.float32)

        def branch(fea, s):
            sm = jnp.max(jnp.where(onehot_b, s, -1e30), axis=0, keepdims=True)
            sm = jnp.where(sm > -1e29, sm, 0.0)
            srow = jnp.sum(onehot * sm, axis=1, keepdims=True)
            e = jnp.exp(s - srow)
            denom = lax.dot_general(e, onehot, (((0,), (0,)), ((), ())),
                                    preferred_element_type=jnp.float32)
            drow = jnp.sum(onehot * denom, axis=1, keepdims=True)
            att = e / (drow + 1e-9)
            bagf = lax.dot_general(onehot, att * fea,
                                   (((0,), (0,)), ((), ())),
                                   preferred_element_type=jnp.float32)
            nrm = jnp.sqrt(jnp.sum(bagf * bagf, axis=1, keepdims=True))
            return att, bagf, bagf / (nrm + 1e-12)

        attq, bagfq, qn = branch(feaq_s[...], sq_s[...])
        attk, _, kn = branch(feak_s[...], sk_s[...])
        attq_ref[...] = attq
        attk_ref[...] = attk
        yprob_ref[...] = jax.nn.sigmoid(
            jnp.dot(bagfq, wcls_ref[...], preferred_element_type=jnp.float32))
        q_ref[...] = qn
        k_ref[...] = kn
        lpos_ref[...] = jnp.sum(qn * kn, axis=1, keepdims=True) / T


def _bank_body(q_ref, lpos_ref, lab_ref, bl_ref, bic_ref, bir_ref, bank_ref,
               logits_ref, nbank_ref, carry):
    j = pl.program_id(0)
    nb = K // COLS

    @pl.when(j < nb)
    def _():
        qm = q_ref[...]
        bank_t = bank_ref[...]
        ln = jnp.dot(qm, bank_t, preferred_element_type=jnp.float32)
        bl = bl_ref[0]
        mask = lab_ref[...] == bl
        ln = jnp.where(mask, -1e9, ln) / T
        head = jnp.where(j == 0, lpos_ref[...], carry[...])
        logits_ref[...] = jnp.concatenate([head, ln[:, :COLS - 1]], axis=1)
        carry[...] = ln[:, COLS - 1:COLS]
        bic = bic_ref[...]
        bir = bir_ref[...]
        ir = lax.broadcasted_iota(jnp.int32, (1, B), 1)
        ic = lax.broadcasted_iota(jnp.int32, (B, 1), 0)
        dup_later = (bic == bir) & (ir > ic)
        is_last = jnp.max(dup_later.astype(jnp.int32), axis=1,
                          keepdims=True) == 0
        cols = lax.broadcasted_iota(jnp.int32, (B, COLS), 1) + j * COLS
        sel = ((bic == cols) & is_last).astype(jnp.float32)
        hit = jnp.max(sel, axis=0, keepdims=True)
        over = lax.dot_general(qm, sel, (((0,), (0,)), ((), ())),
                               preferred_element_type=jnp.float32)
        nbank_ref[...] = bank_t * (1.0 - hit) + over

    @pl.when(j == nb)
    def _():
        logits_ref[:, 0:1] = carry[...]


def kernel(im_q, im_k, batch, bag_idx, label, bag_label, W_enc_q, W_self_q,
           V_q, U_q, w_att_q, W_cls_q, W_enc_k, W_self_k, bank):
    f32 = jnp.float32
    wq_cat = jnp.concatenate([W_enc_q, W_self_q], axis=1)
    wk_cat = jnp.concatenate([W_enc_k, W_self_k], axis=1)

    n_row_blocks = N_INST // ROWS
    (sfq, sfk, sq, sk, attq, attk, yprob, qn, kn, lpos) = pl.pallas_call(
        _enc_agg_body,
        grid=(n_row_blocks,),
        in_specs=[
            pl.BlockSpec((ROWS, D_IN), lambda i: (i, 0)),
            pl.BlockSpec((ROWS, D_IN), lambda i: (i, 0)),
            pl.BlockSpec((D_IN, 2 * DIM), lambda i: (0, 0)),
            pl.BlockSpec((D_IN, 2 * DIM), lambda i: (0, 0)),
            pl.BlockSpec((DIM, DIM), lambda i: (0, 0)),
            pl.BlockSpec((DIM, DIM), lambda i: (0, 0)),
            pl.BlockSpec((DIM, 1), lambda i: (0, 0)),
            pl.BlockSpec((N_INST, 1), lambda i: (0, 0)),
            pl.BlockSpec((DIM, 1), lambda i: (0, 0)),
        ],
        out_specs=[
            pl.BlockSpec((ROWS, DIM), lambda i: (i, 0)),
            pl.BlockSpec((ROWS, DIM), lambda i: (i, 0)),
            pl.BlockSpec((ROWS, 1), lambda i: (i, 0)),
            pl.BlockSpec((ROWS, 1), lambda i: (i, 0)),
            pl.BlockSpec((N_INST, 1), lambda i: (0, 0)),
            pl.BlockSpec((N_INST, 1), lambda i: (0, 0)),
            pl.BlockSpec((B, 1), lambda i: (0, 0)),
            pl.BlockSpec((B, DIM), lambda i: (0, 0)),
            pl.BlockSpec((B, DIM), lambda i: (0, 0)),
            pl.BlockSpec((B, 1), lambda i: (0, 0)),
        ],
        out_shape=[
            jax.ShapeDtypeStruct((N_INST, DIM), f32),
            jax.ShapeDtypeStruct((N_INST, DIM), f32),
            jax.ShapeDtypeStruct((N_INST, 1), f32),
            jax.ShapeDtypeStruct((N_INST, 1), f32),
            jax.ShapeDtypeStruct((N_INST, 1), f32),
            jax.ShapeDtypeStruct((N_INST, 1), f32),
            jax.ShapeDtypeStruct((B, 1), f32),
            jax.ShapeDtypeStruct((B, DIM), f32),
            jax.ShapeDtypeStruct((B, DIM), f32),
            jax.ShapeDtypeStruct((B, 1), f32),
        ],
        scratch_shapes=[
            pltpu.VMEM((N_INST, DIM), f32),
            pltpu.VMEM((N_INST, DIM), f32),
            pltpu.VMEM((N_INST, 1), f32),
            pltpu.VMEM((N_INST, 1), f32),
        ],
    )(im_q, im_k, wq_cat, wk_cat, V_q, U_q, w_att_q,
      batch.reshape(N_INST, 1).astype(jnp.int32), W_cls_q)

    n_col_blocks = K // COLS
    last = n_col_blocks - 1
    logits, nbank = pl.pallas_call(
        _bank_body,
        grid=(n_col_blocks + 1,),
        in_specs=[
            pl.BlockSpec((B, DIM), lambda j: (0, 0)),
            pl.BlockSpec((B, 1), lambda j: (0, 0)),
            pl.BlockSpec((B, 1), lambda j: (0, 0)),
            pl.BlockSpec((1, 1, COLS), lambda j: (jnp.minimum(j, last), 0, 0)),
            pl.BlockSpec((B, 1), lambda j: (0, 0)),
            pl.BlockSpec((1, B), lambda j: (0, 0)),
            pl.BlockSpec((DIM, COLS), lambda j: (0, jnp.minimum(j, last))),
        ],
        out_specs=[
            pl.BlockSpec((B, COLS), lambda j: (0, j)),
            pl.BlockSpec((DIM, COLS), lambda j: (0, jnp.minimum(j, last))),
        ],
        out_shape=[
            jax.ShapeDtypeStruct((B, K + 1), f32),
            jax.ShapeDtypeStruct((DIM, K), f32),
        ],
        scratch_shapes=[
            pltpu.VMEM((B, 1), f32),
        ],
    )(qn, lpos, label.reshape(B, 1).astype(jnp.int32),
      bag_label.reshape(n_col_blocks, 1, COLS).astype(jnp.int32),
      bag_idx.reshape(B, 1).astype(jnp.int32),
      bag_idx.reshape(1, B).astype(jnp.int32), bank)

    labels = jnp.zeros((B,), jnp.int32)
    return (yprob, logits, labels, nbank, sfq, sfk,
            attq.reshape(N_INST), attk.reshape(N_INST),
            sq.reshape(N_INST), sk.reshape(N_INST))


# speedup vs baseline: 7.6047x; 1.0026x over previous
import jax
import jax.numpy as jnp
from jax import lax
from jax.experimental import pallas as pl
from jax.experimental.pallas import tpu as pltpu

N_INST = 8192
D_IN = 1024
DIM = 128
B = 128
K = 65536
T = 0.07
EMA = 0.999

ROWS = 512
COLS = 16384


def _enc_agg_body(imq_ref, imk_ref, wq_ref, wk_ref, v_ref, u_ref, wa_ref,
                  batch_ref, wcls_ref,
                  sfq_ref, sfk_ref, sqo_ref, sko_ref,
                  attq_ref, attk_ref, yprob_ref, q_ref, k_ref, lpos_ref,
                  feaq_s, feak_s, sq_s, sk_s):
    i = pl.program_id(0)
    wq = wq_ref[...]
    wk = EMA * wk_ref[...] + (1.0 - EMA) * wq
    hq = jnp.dot(imq_ref[...], wq, preferred_element_type=jnp.float32)
    hk = jnp.dot(imk_ref[...], wk, preferred_element_type=jnp.float32)
    feaq = jnp.tanh(hq[:, :DIM])
    sfq = jnp.tanh(hq[:, DIM:])
    feak = jnp.tanh(hk[:, :DIM])
    sfk = jnp.tanh(hk[:, DIM:])
    sfq_ref[...] = sfq
    sfk_ref[...] = sfk
    v = v_ref[...]
    u = u_ref[...]
    wa = wa_ref[...]
    aq = jnp.tanh(jnp.dot(feaq, v, preferred_element_type=jnp.float32)) * \
        jax.nn.sigmoid(jnp.dot(feaq, u, preferred_element_type=jnp.float32))
    ak = jnp.tanh(jnp.dot(feak, v, preferred_element_type=jnp.float32)) * \
        jax.nn.sigmoid(jnp.dot(feak, u, preferred_element_type=jnp.float32))
    sq = jnp.dot(aq, wa, preferred_element_type=jnp.float32)
    sk = jnp.dot(ak, wa, preferred_element_type=jnp.float32)
    sqo_ref[...] = sq
    sko_ref[...] = sk
    base = i * ROWS
    feaq_s[pl.ds(base, ROWS), :] = feaq
    feak_s[pl.ds(base, ROWS), :] = feak
    sq_s[pl.ds(base, ROWS), :] = sq
    sk_s[pl.ds(base, ROWS), :] = sk

    @pl.when(i == (N_INST // ROWS) - 1)
    def _():
        batch = batch_ref[...]
        seg = lax.broadcasted_iota(jnp.int32, (1, B), 1)
        onehot_b = batch == seg
        onehot = onehot_b.astype(jnp.float32)

        def branch(fea, s):
            sm = jnp.max(jnp.where(onehot_b, s, -1e30), axis=0, keepdims=True)
            sm = jnp.where(sm > -1e29, sm, 0.0)
            srow = jnp.sum(onehot * sm, axis=1, keepdims=True)
            e = jnp.exp(s - srow)
            denom = lax.dot_general(e, onehot, (((0,), (0,)), ((), ())),
                                    preferred_element_type=jnp.float32)
            drow = jnp.sum(onehot * denom, axis=1, keepdims=True)
            att = e / (drow + 1e-9)
            bagf = lax.dot_general(onehot, att * fea,
                                   (((0,), (0,)), ((), ())),
                                   preferred_element_type=jnp.float32)
            nrm = jnp.sqrt(jnp.sum(bagf * bagf, axis=1, keepdims=True))
            return att, bagf, bagf / (nrm + 1e-12)

        attq, bagfq, qn = branch(feaq_s[...], sq_s[...])
        attk, _, kn = branch(feak_s[...], sk_s[...])
        attq_ref[...] = attq
        attk_ref[...] = attk
        yprob_ref[...] = jax.nn.sigmoid(
            jnp.dot(bagfq, wcls_ref[...], preferred_element_type=jnp.float32))
        q_ref[...] = qn
        k_ref[...] = kn
        lpos_ref[...] = jnp.sum(qn * kn, axis=1, keepdims=True) / T


def _bank_body(q_ref, lpos_ref, lab_ref, bl_ref, bic_ref, bir_ref, bank_ref,
               logits_ref, nbank_ref, carry):
    j = pl.program_id(0)
    nb = K // COLS

    @pl.when(j < nb)
    def _():
        qm = q_ref[...]
        bank_t = bank_ref[...]
        ln = jnp.dot(qm, bank_t, preferred_element_type=jnp.float32)
        bl = bl_ref[0]
        mask = lab_ref[...] == bl
        ln = jnp.where(mask, -1e9, ln) / T
        head = jnp.where(j == 0, lpos_ref[...], carry[...])
        logits_ref[...] = jnp.concatenate([head, ln[:, :COLS - 1]], axis=1)
        carry[...] = ln[:, COLS - 1:COLS]
        bic = bic_ref[...]
        bir = bir_ref[...]
        ir = lax.broadcasted_iota(jnp.int32, (1, B), 1)
        ic = lax.broadcasted_iota(jnp.int32, (B, 1), 0)
        dup_later = (bic == bir) & (ir > ic)
        is_last = jnp.max(dup_later.astype(jnp.int32), axis=1,
                          keepdims=True) == 0
        cols = lax.broadcasted_iota(jnp.int32, (B, COLS), 1) + j * COLS
        sel = ((bic == cols) & is_last).astype(jnp.float32)
        hit = jnp.max(sel, axis=0, keepdims=True)
        over = lax.dot_general(qm, sel, (((0,), (0,)), ((), ())),
                               preferred_element_type=jnp.float32)
        nbank_ref[...] = bank_t * (1.0 - hit) + over

    @pl.when(j == nb)
    def _():
        logits_ref[:, 0:1] = carry[...]


def kernel(im_q, im_k, batch, bag_idx, label, bag_label, W_enc_q, W_self_q,
           V_q, U_q, w_att_q, W_cls_q, W_enc_k, W_self_k, bank):
    f32 = jnp.float32
    wq_cat = jnp.concatenate([W_enc_q, W_self_q], axis=1)
    wk_cat = jnp.concatenate([W_enc_k, W_self_k], axis=1)

    n_row_blocks = N_INST // ROWS
    (sfq, sfk, sq, sk, attq, attk, yprob, qn, kn, lpos) = pl.pallas_call(
        _enc_agg_body,
        grid=(n_row_blocks,),
        in_specs=[
            pl.BlockSpec((ROWS, D_IN), lambda i: (i, 0)),
            pl.BlockSpec((ROWS, D_IN), lambda i: (i, 0)),
            pl.BlockSpec((D_IN, 2 * DIM), lambda i: (0, 0)),
            pl.BlockSpec((D_IN, 2 * DIM), lambda i: (0, 0)),
            pl.BlockSpec((DIM, DIM), lambda i: (0, 0)),
            pl.BlockSpec((DIM, DIM), lambda i: (0, 0)),
            pl.BlockSpec((DIM, 1), lambda i: (0, 0)),
            pl.BlockSpec((N_INST, 1), lambda i: (0, 0)),
            pl.BlockSpec((DIM, 1), lambda i: (0, 0)),
        ],
        out_specs=[
            pl.BlockSpec((ROWS, DIM), lambda i: (i, 0)),
            pl.BlockSpec((ROWS, DIM), lambda i: (i, 0)),
            pl.BlockSpec((ROWS, 1), lambda i: (i, 0)),
            pl.BlockSpec((ROWS, 1), lambda i: (i, 0)),
            pl.BlockSpec((N_INST, 1), lambda i: (0, 0)),
            pl.BlockSpec((N_INST, 1), lambda i: (0, 0)),
            pl.BlockSpec((B, 1), lambda i: (0, 0)),
            pl.BlockSpec((B, DIM), lambda i: (0, 0)),
            pl.BlockSpec((B, DIM), lambda i: (0, 0)),
            pl.BlockSpec((B, 1), lambda i: (0, 0)),
        ],
        out_shape=[
            jax.ShapeDtypeStruct((N_INST, DIM), f32),
            jax.ShapeDtypeStruct((N_INST, DIM), f32),
            jax.ShapeDtypeStruct((N_INST, 1), f32),
            jax.ShapeDtypeStruct((N_INST, 1), f32),
            jax.ShapeDtypeStruct((N_INST, 1), f32),
            jax.ShapeDtypeStruct((N_INST, 1), f32),
            jax.ShapeDtypeStruct((B, 1), f32),
            jax.ShapeDtypeStruct((B, DIM), f32),
            jax.ShapeDtypeStruct((B, DIM), f32),
            jax.ShapeDtypeStruct((B, 1), f32),
        ],
        scratch_shapes=[
            pltpu.VMEM((N_INST, DIM), f32),
            pltpu.VMEM((N_INST, DIM), f32),
            pltpu.VMEM((N_INST, 1), f32),
            pltpu.VMEM((N_INST, 1), f32),
        ],
    )(im_q, im_k, wq_cat, wk_cat, V_q, U_q, w_att_q,
      batch.reshape(N_INST, 1).astype(jnp.int32), W_cls_q)

    n_col_blocks = K // COLS
    last = n_col_blocks - 1
    logits, nbank = pl.pallas_call(
        _bank_body,
        grid=(n_col_blocks + 1,),
        in_specs=[
            pl.BlockSpec((B, DIM), lambda j: (0, 0)),
            pl.BlockSpec((B, 1), lambda j: (0, 0)),
            pl.BlockSpec((B, 1), lambda j: (0, 0)),
            pl.BlockSpec((1, 1, COLS), lambda j: (jnp.minimum(j, last), 0, 0)),
            pl.BlockSpec((B, 1), lambda j: (0, 0)),
            pl.BlockSpec((1, B), lambda j: (0, 0)),
            pl.BlockSpec((DIM, COLS), lambda j: (0, jnp.minimum(j, last))),
        ],
        out_specs=[
            pl.BlockSpec((B, COLS), lambda j: (0, j)),
            pl.BlockSpec((DIM, COLS), lambda j: (0, jnp.minimum(j, last))),
        ],
        out_shape=[
            jax.ShapeDtypeStruct((B, K + 1), f32),
            jax.ShapeDtypeStruct((DIM, K), f32),
        ],
        scratch_shapes=[
            pltpu.VMEM((B, 1), f32),
        ],
    )(qn, lpos, label.reshape(B, 1).astype(jnp.int32),
      bag_label.reshape(n_col_blocks, 1, COLS).astype(jnp.int32),
      bag_idx.reshape(B, 1).astype(jnp.int32),
      bag_idx.reshape(1, B).astype(jnp.int32), bank)

    labels = jnp.zeros((B,), jnp.int32)
    return (yprob, logits, labels, nbank, sfq, sfk,
            attq.reshape(N_INST), attk.reshape(N_INST),
            sq.reshape(N_INST), sk.reshape(N_INST))
